# Initial kernel scaffold; baseline (speedup 1.0000x reference)
import jax
import jax.numpy as jnp
from jax.experimental import pallas as pl


def kernel(node_features, batch, lin_W, lin_b, att_W, att_b, le1_W, le1_b, le2_W, le3_W, le3_b):
    raise NotImplementedError("write your pallas kernel here")


# speedup vs baseline: 1.8410x; 1.8410x over previous
import math
from functools import partial

import jax
import jax.numpy as jnp
from jax.experimental import pallas as pl

N = 8192
F = 128
K = 10
NKEEP = 4096
ROWB = 256
EB = 2048


def _bf(v):
    return v.astype(jnp.bfloat16)


def _dist_topk_body(x_ref, xt_ref, dcol_ref, drow_ref, nbr_ref):
    g = jnp.dot(_bf(x_ref[...]), _bf(xt_ref[...]), preferred_element_type=jnp.float32)
    dist = dcol_ref[...] - 2.0 * g + drow_ref[...]
    lane = jax.lax.broadcasted_iota(jnp.int32, (ROWB, N), 1)
    big = jnp.int32(2**30)
    for k in range(K):
        m = jnp.min(dist, axis=1, keepdims=True)
        idx = jnp.min(jnp.where(dist == m, lane, big), axis=1, keepdims=True)
        nbr_ref[:, k : k + 1] = idx
        dist = jnp.where(lane == idx, jnp.inf, dist)
    nbr_ref[:, K:] = jnp.zeros((ROWB, 16 - K), jnp.int32)


def _dist_topk(x, xt, d):
    grid = (N // ROWB,)
    return pl.pallas_call(
        _dist_topk_body,
        grid=grid,
        in_specs=[
            pl.BlockSpec((ROWB, F), lambda i: (i, 0)),
            pl.BlockSpec((F, N), lambda i: (0, 0)),
            pl.BlockSpec((ROWB, 1), lambda i: (i, 0)),
            pl.BlockSpec((1, N), lambda i: (0, 0)),
        ],
        out_specs=pl.BlockSpec((ROWB, 16), lambda i: (i, 0)),
        out_shape=jax.ShapeDtypeStruct((N, 16), jnp.int32),
    )(x, xt, d.reshape(N, 1), d.reshape(1, N))


def _q_body(xq_ref, w_ref, b_ref, o_ref):
    o_ref[...] = (
        jnp.dot(_bf(xq_ref[...]), _bf(w_ref[...]), preferred_element_type=jnp.float32)
        + b_ref[...]
    )


def _q_matmul(x_q, lin_W, lin_b):
    return pl.pallas_call(
        _q_body,
        grid=(8,),
        in_specs=[
            pl.BlockSpec((N // 8, F), lambda i: (i, 0)),
            pl.BlockSpec((F, F), lambda i: (0, 0)),
            pl.BlockSpec((1, F), lambda i: (0, 0)),
        ],
        out_specs=pl.BlockSpec((N // 8, F), lambda i: (i, 0)),
        out_shape=jax.ShapeDtypeStruct((N, F), jnp.float32),
    )(x_q, lin_W, lin_b.reshape(1, F))


def _score_body(qd_ref, xs_ref, w_ref, b_ref, o_ref):
    ef = jnp.concatenate([qd_ref[...], xs_ref[...]], axis=1)
    s = jnp.dot(_bf(ef), _bf(w_ref[...]), preferred_element_type=jnp.float32) + b_ref[...]
    o_ref[...] = jnp.where(s >= 0, s, 0.2 * s)


def _score(q_dst, x_src, att_W, att_b):
    E = N * K
    return pl.pallas_call(
        _score_body,
        grid=(E // EB,),
        in_specs=[
            pl.BlockSpec((EB, F), lambda i: (i, 0)),
            pl.BlockSpec((EB, F), lambda i: (i, 0)),
            pl.BlockSpec((2 * F, 1), lambda i: (0, 0)),
            pl.BlockSpec((1, 1), lambda i: (0, 0)),
        ],
        out_specs=pl.BlockSpec((EB, 1), lambda i: (i, 0)),
        out_shape=jax.ShapeDtypeStruct((E, 1), jnp.float32),
    )(q_dst, x_src, att_W, att_b.reshape(1, 1))


def _exp_body(s_ref, m_ref, o_ref):
    o_ref[...] = jnp.exp(s_ref[...] - m_ref[...])


def _edge_exp(score, smax_d):
    E = N * K
    return pl.pallas_call(
        _exp_body,
        grid=(E // EB,),
        in_specs=[
            pl.BlockSpec((EB, 1), lambda i: (i, 0)),
            pl.BlockSpec((EB, 1), lambda i: (i, 0)),
        ],
        out_specs=pl.BlockSpec((EB, 1), lambda i: (i, 0)),
        out_shape=jax.ShapeDtypeStruct((E, 1), jnp.float32),
    )(score, smax_d)


def _vj_body(e_ref, sd_ref, xs_ref, o_ref):
    w = e_ref[...] / (sd_ref[...] + 1e-16)
    o_ref[...] = xs_ref[...] * w


def _vj(e, ssum_d, x_src):
    E = N * K
    return pl.pallas_call(
        _vj_body,
        grid=(E // EB,),
        in_specs=[
            pl.BlockSpec((EB, 1), lambda i: (i, 0)),
            pl.BlockSpec((EB, 1), lambda i: (i, 0)),
            pl.BlockSpec((EB, F), lambda i: (i, 0)),
        ],
        out_specs=pl.BlockSpec((EB, F), lambda i: (i, 0)),
        out_shape=jax.ShapeDtypeStruct((E, F), jnp.float32),
    )(e, ssum_d, x_src)


def _le_body(xn_ref, w1_ref, b1_ref, w2_ref, w3_ref, a_ref, b_ref, c_ref):
    xb = _bf(xn_ref[...])
    a_ref[...] = jnp.dot(xb, _bf(w1_ref[...]), preferred_element_type=jnp.float32) + b1_ref[...]
    b_ref[...] = jnp.dot(xb, _bf(w2_ref[...]), preferred_element_type=jnp.float32)
    c_ref[...] = jnp.dot(xb, _bf(w3_ref[...]), preferred_element_type=jnp.float32)


def _le_matvecs(x_new, le1_W, le1_b, le2_W, le3_W):
    return pl.pallas_call(
        _le_body,
        grid=(8,),
        in_specs=[
            pl.BlockSpec((N // 8, F), lambda i: (i, 0)),
            pl.BlockSpec((F, 1), lambda i: (0, 0)),
            pl.BlockSpec((1, 1), lambda i: (0, 0)),
            pl.BlockSpec((F, 1), lambda i: (0, 0)),
            pl.BlockSpec((F, 1), lambda i: (0, 0)),
        ],
        out_specs=[
            pl.BlockSpec((N // 8, 1), lambda i: (i, 0)),
            pl.BlockSpec((N // 8, 1), lambda i: (i, 0)),
            pl.BlockSpec((N // 8, 1), lambda i: (i, 0)),
        ],
        out_shape=[
            jax.ShapeDtypeStruct((N, 1), jnp.float32),
            jax.ShapeDtypeStruct((N, 1), jnp.float32),
            jax.ShapeDtypeStruct((N, 1), jnp.float32),
        ],
    )(x_new, le1_W, le1_b.reshape(1, 1), le2_W, le3_W)


def _msg_body(ad_ref, bs_ref, o_ref):
    o_ref[...] = ad_ref[...] - bs_ref[...]


def _msg(a_d, b_s):
    E = N * K
    return pl.pallas_call(
        _msg_body,
        grid=(E // EB,),
        in_specs=[
            pl.BlockSpec((EB, 1), lambda i: (i, 0)),
            pl.BlockSpec((EB, 1), lambda i: (i, 0)),
        ],
        out_specs=pl.BlockSpec((EB, 1), lambda i: (i, 0)),
        out_shape=jax.ShapeDtypeStruct((E, 1), jnp.float32),
    )(a_d, b_s)


def _fit_body(agg_ref, c_ref, b3_ref, o_ref):
    o_ref[...] = jax.nn.sigmoid((agg_ref[...] + c_ref[...]) + b3_ref[...])


def _fitness(agg, xle3, le3_b):
    return pl.pallas_call(
        _fit_body,
        grid=(8,),
        in_specs=[
            pl.BlockSpec((N // 8, 1), lambda i: (i, 0)),
            pl.BlockSpec((N // 8, 1), lambda i: (i, 0)),
            pl.BlockSpec((1, 1), lambda i: (0, 0)),
        ],
        out_specs=pl.BlockSpec((N // 8, 1), lambda i: (i, 0)),
        out_shape=jax.ShapeDtypeStruct((N, 1), jnp.float32),
    )(agg, xle3, le3_b.reshape(1, 1))


def _rank_body(fblk_ref, fall_ref, rank_ref):
    fb = fblk_ref[...]
    fa = fall_ref[...]
    i_blk = (
        jax.lax.broadcasted_iota(jnp.int32, (ROWB, 1), 0)
        + pl.program_id(0) * ROWB
    )
    j = jax.lax.broadcasted_iota(jnp.int32, (ROWB, N), 1)
    gt = (fa > fb).astype(jnp.int32)
    eq = ((fa == fb) & (j < i_blk)).astype(jnp.int32)
    rank_ref[...] = jnp.sum(gt + eq, axis=1, keepdims=True)


def _rank(fitness):
    return pl.pallas_call(
        _rank_body,
        grid=(N // ROWB,),
        in_specs=[
            pl.BlockSpec((ROWB, 1), lambda i: (i, 0)),
            pl.BlockSpec((1, N), lambda i: (0, 0)),
        ],
        out_specs=pl.BlockSpec((ROWB, 1), lambda i: (i, 0)),
        out_shape=jax.ShapeDtypeStruct((N, 1), jnp.int32),
    )(fitness, fitness.reshape(1, N))


def _perm_body(rank_ref, fall_ref, perm_ref, fsel_ref):
    r_blk = (
        jax.lax.broadcasted_iota(jnp.int32, (ROWB, 1), 0)
        + pl.program_id(0) * ROWB
    )
    ranks = rank_ref[...]
    fall = fall_ref[...]
    i = jax.lax.broadcasted_iota(jnp.int32, (ROWB, N), 1)
    hit = ranks == r_blk
    perm_ref[...] = jnp.sum(jnp.where(hit, i, 0), axis=1, keepdims=True)
    fsel_ref[...] = jnp.sum(jnp.where(hit, fall, 0.0), axis=1, keepdims=True)


def _perm(rank, fitness):
    return pl.pallas_call(
        _perm_body,
        grid=(NKEEP // ROWB,),
        in_specs=[
            pl.BlockSpec((1, N), lambda i: (0, 0)),
            pl.BlockSpec((1, N), lambda i: (0, 0)),
        ],
        out_specs=[
            pl.BlockSpec((ROWB, 1), lambda i: (i, 0)),
            pl.BlockSpec((ROWB, 1), lambda i: (i, 0)),
        ],
        out_shape=[
            jax.ShapeDtypeStruct((NKEEP, 1), jnp.int32),
            jax.ShapeDtypeStruct((NKEEP, 1), jnp.float32),
        ],
    )(rank.reshape(1, N), fitness.reshape(1, N))


def _scale_body(rows_ref, f_ref, o_ref):
    o_ref[...] = rows_ref[...] * f_ref[...]


def _scale(rows, fsel):
    return pl.pallas_call(
        _scale_body,
        grid=(NKEEP // EB if NKEEP >= EB else 1,),
        in_specs=[
            pl.BlockSpec((min(EB, NKEEP), F), lambda i: (i, 0)),
            pl.BlockSpec((min(EB, NKEEP), 1), lambda i: (i, 0)),
        ],
        out_specs=pl.BlockSpec((min(EB, NKEEP), F), lambda i: (i, 0)),
        out_shape=jax.ShapeDtypeStruct((NKEEP, F), jnp.float32),
    )(rows, fsel)


def kernel(node_features, batch, lin_W, lin_b, att_W, att_b, le1_W, le1_b, le2_W, le3_W, le3_b):
    x = node_features
    n = x.shape[0]
    d = jnp.sum(x * x, axis=1)
    nbr = _dist_topk(x, x.T, d)[:, :K]
    src = jnp.repeat(jnp.arange(n), K)
    dst = nbr.reshape(-1)
    x_src = x[src]
    x_q = jax.ops.segment_max(x_src, dst, num_segments=n)
    x_q = jnp.where(jnp.isfinite(x_q), x_q, 0.0)
    q = _q_matmul(x_q, lin_W, lin_b)
    score = _score(q[dst], x_src, att_W, att_b).reshape(-1)
    smax = jax.ops.segment_max(score, dst, num_segments=n)
    smax = jnp.where(jnp.isfinite(smax), smax, 0.0)
    e = _edge_exp(score.reshape(-1, 1), smax[dst].reshape(-1, 1)).reshape(-1)
    ssum = jax.ops.segment_sum(e, dst, num_segments=n)
    v_j = _vj(e.reshape(-1, 1), ssum[dst].reshape(-1, 1), x_src)
    x_new = jax.ops.segment_sum(v_j, dst, num_segments=n)
    a, b, xle3 = _le_matvecs(x_new, le1_W, le1_b, le2_W, le3_W)
    msg = _msg(a[dst], b[src])
    agg = jax.ops.segment_sum(msg, dst, num_segments=n)
    fitness = _fitness(agg, xle3, le3_b).reshape(-1)
    rank = _rank(fitness.reshape(-1, 1))
    perm, fsel = _perm(rank, fitness)
    perm = perm.reshape(-1)
    out = _scale(x_new[perm], fsel)
    new_batch = batch[perm]
    return out, new_batch


# speedup vs baseline: 1.8557x; 1.0080x over previous
import math
from functools import partial

import jax
import jax.numpy as jnp
from jax.experimental import pallas as pl

N = 8192
F = 128
K = 10
NKEEP = 4096
ROWB = 256
EB = 2048


def _bf(v):
    return v.astype(jnp.bfloat16)


def _dist_topk_body(x_ref, xt_ref, dcol_ref, drow_ref, nbr_ref):
    g = jnp.dot(_bf(x_ref[...]), _bf(xt_ref[...]), preferred_element_type=jnp.float32)
    dist = dcol_ref[...] - 2.0 * g + drow_ref[...]
    lane = jax.lax.broadcasted_iota(jnp.int32, (ROWB, N), 1)
    for k in range(K):
        idx = jnp.argmin(dist, axis=1).astype(jnp.int32).reshape(ROWB, 1)
        nbr_ref[:, k : k + 1] = idx
        dist = jnp.where(lane == idx, jnp.inf, dist)
    nbr_ref[:, K:] = jnp.zeros((ROWB, 16 - K), jnp.int32)


def _dist_topk(x, xt, d):
    grid = (N // ROWB,)
    return pl.pallas_call(
        _dist_topk_body,
        grid=grid,
        in_specs=[
            pl.BlockSpec((ROWB, F), lambda i: (i, 0)),
            pl.BlockSpec((F, N), lambda i: (0, 0)),
            pl.BlockSpec((ROWB, 1), lambda i: (i, 0)),
            pl.BlockSpec((1, N), lambda i: (0, 0)),
        ],
        out_specs=pl.BlockSpec((ROWB, 16), lambda i: (i, 0)),
        out_shape=jax.ShapeDtypeStruct((N, 16), jnp.int32),
    )(x, xt, d.reshape(N, 1), d.reshape(1, N))


def _q_body(xq_ref, w_ref, b_ref, o_ref):
    o_ref[...] = (
        jnp.dot(_bf(xq_ref[...]), _bf(w_ref[...]), preferred_element_type=jnp.float32)
        + b_ref[...]
    )


def _q_matmul(x_q, lin_W, lin_b):
    return pl.pallas_call(
        _q_body,
        grid=(8,),
        in_specs=[
            pl.BlockSpec((N // 8, F), lambda i: (i, 0)),
            pl.BlockSpec((F, F), lambda i: (0, 0)),
            pl.BlockSpec((1, F), lambda i: (0, 0)),
        ],
        out_specs=pl.BlockSpec((N // 8, F), lambda i: (i, 0)),
        out_shape=jax.ShapeDtypeStruct((N, F), jnp.float32),
    )(x_q, lin_W, lin_b.reshape(1, F))


def _score_body(qd_ref, xs_ref, w_ref, b_ref, o_ref):
    ef = jnp.concatenate([qd_ref[...], xs_ref[...]], axis=1)
    s = jnp.dot(_bf(ef), _bf(w_ref[...]), preferred_element_type=jnp.float32) + b_ref[...]
    o_ref[...] = jnp.where(s >= 0, s, 0.2 * s)


def _score(q_dst, x_src, att_W, att_b):
    E = N * K
    return pl.pallas_call(
        _score_body,
        grid=(E // EB,),
        in_specs=[
            pl.BlockSpec((EB, F), lambda i: (i, 0)),
            pl.BlockSpec((EB, F), lambda i: (i, 0)),
            pl.BlockSpec((2 * F, 1), lambda i: (0, 0)),
            pl.BlockSpec((1, 1), lambda i: (0, 0)),
        ],
        out_specs=pl.BlockSpec((EB, 1), lambda i: (i, 0)),
        out_shape=jax.ShapeDtypeStruct((E, 1), jnp.float32),
    )(q_dst, x_src, att_W, att_b.reshape(1, 1))


def _exp_body(s_ref, m_ref, o_ref):
    o_ref[...] = jnp.exp(s_ref[...] - m_ref[...])


def _edge_exp(score, smax_d):
    E = N * K
    return pl.pallas_call(
        _exp_body,
        grid=(E // EB,),
        in_specs=[
            pl.BlockSpec((EB, 1), lambda i: (i, 0)),
            pl.BlockSpec((EB, 1), lambda i: (i, 0)),
        ],
        out_specs=pl.BlockSpec((EB, 1), lambda i: (i, 0)),
        out_shape=jax.ShapeDtypeStruct((E, 1), jnp.float32),
    )(score, smax_d)


def _vj_body(e_ref, sd_ref, xs_ref, o_ref):
    w = e_ref[...] / (sd_ref[...] + 1e-16)
    o_ref[...] = xs_ref[...] * w


def _vj(e, ssum_d, x_src):
    E = N * K
    return pl.pallas_call(
        _vj_body,
        grid=(E // EB,),
        in_specs=[
            pl.BlockSpec((EB, 1), lambda i: (i, 0)),
            pl.BlockSpec((EB, 1), lambda i: (i, 0)),
            pl.BlockSpec((EB, F), lambda i: (i, 0)),
        ],
        out_specs=pl.BlockSpec((EB, F), lambda i: (i, 0)),
        out_shape=jax.ShapeDtypeStruct((E, F), jnp.float32),
    )(e, ssum_d, x_src)


def _le_body(xn_ref, w1_ref, b1_ref, w2_ref, w3_ref, a_ref, b_ref, c_ref):
    xb = _bf(xn_ref[...])
    a_ref[...] = jnp.dot(xb, _bf(w1_ref[...]), preferred_element_type=jnp.float32) + b1_ref[...]
    b_ref[...] = jnp.dot(xb, _bf(w2_ref[...]), preferred_element_type=jnp.float32)
    c_ref[...] = jnp.dot(xb, _bf(w3_ref[...]), preferred_element_type=jnp.float32)


def _le_matvecs(x_new, le1_W, le1_b, le2_W, le3_W):
    return pl.pallas_call(
        _le_body,
        grid=(8,),
        in_specs=[
            pl.BlockSpec((N // 8, F), lambda i: (i, 0)),
            pl.BlockSpec((F, 1), lambda i: (0, 0)),
            pl.BlockSpec((1, 1), lambda i: (0, 0)),
            pl.BlockSpec((F, 1), lambda i: (0, 0)),
            pl.BlockSpec((F, 1), lambda i: (0, 0)),
        ],
        out_specs=[
            pl.BlockSpec((N // 8, 1), lambda i: (i, 0)),
            pl.BlockSpec((N // 8, 1), lambda i: (i, 0)),
            pl.BlockSpec((N // 8, 1), lambda i: (i, 0)),
        ],
        out_shape=[
            jax.ShapeDtypeStruct((N, 1), jnp.float32),
            jax.ShapeDtypeStruct((N, 1), jnp.float32),
            jax.ShapeDtypeStruct((N, 1), jnp.float32),
        ],
    )(x_new, le1_W, le1_b.reshape(1, 1), le2_W, le3_W)


def _msg_body(ad_ref, bs_ref, o_ref):
    o_ref[...] = ad_ref[...] - bs_ref[...]


def _msg(a_d, b_s):
    E = N * K
    return pl.pallas_call(
        _msg_body,
        grid=(E // EB,),
        in_specs=[
            pl.BlockSpec((EB, 1), lambda i: (i, 0)),
            pl.BlockSpec((EB, 1), lambda i: (i, 0)),
        ],
        out_specs=pl.BlockSpec((EB, 1), lambda i: (i, 0)),
        out_shape=jax.ShapeDtypeStruct((E, 1), jnp.float32),
    )(a_d, b_s)


def _fit_body(agg_ref, c_ref, b3_ref, o_ref):
    o_ref[...] = jax.nn.sigmoid((agg_ref[...] + c_ref[...]) + b3_ref[...])


def _fitness(agg, xle3, le3_b):
    return pl.pallas_call(
        _fit_body,
        grid=(8,),
        in_specs=[
            pl.BlockSpec((N // 8, 1), lambda i: (i, 0)),
            pl.BlockSpec((N // 8, 1), lambda i: (i, 0)),
            pl.BlockSpec((1, 1), lambda i: (0, 0)),
        ],
        out_specs=pl.BlockSpec((N // 8, 1), lambda i: (i, 0)),
        out_shape=jax.ShapeDtypeStruct((N, 1), jnp.float32),
    )(agg, xle3, le3_b.reshape(1, 1))


def _rank_body(fblk_ref, fall_ref, rank_ref):
    fb = fblk_ref[...]
    fa = fall_ref[...]
    i_blk = (
        jax.lax.broadcasted_iota(jnp.int32, (ROWB, 1), 0)
        + pl.program_id(0) * ROWB
    )
    j = jax.lax.broadcasted_iota(jnp.int32, (ROWB, N), 1)
    gt = (fa > fb).astype(jnp.int32)
    eq = ((fa == fb) & (j < i_blk)).astype(jnp.int32)
    rank_ref[...] = jnp.sum(gt + eq, axis=1, keepdims=True)


def _rank(fitness):
    return pl.pallas_call(
        _rank_body,
        grid=(N // ROWB,),
        in_specs=[
            pl.BlockSpec((ROWB, 1), lambda i: (i, 0)),
            pl.BlockSpec((1, N), lambda i: (0, 0)),
        ],
        out_specs=pl.BlockSpec((ROWB, 1), lambda i: (i, 0)),
        out_shape=jax.ShapeDtypeStruct((N, 1), jnp.int32),
    )(fitness, fitness.reshape(1, N))


def _perm_body(rank_ref, fall_ref, perm_ref, fsel_ref):
    r_blk = (
        jax.lax.broadcasted_iota(jnp.int32, (ROWB, 1), 0)
        + pl.program_id(0) * ROWB
    )
    ranks = rank_ref[...]
    fall = fall_ref[...]
    i = jax.lax.broadcasted_iota(jnp.int32, (ROWB, N), 1)
    hit = ranks == r_blk
    perm_ref[...] = jnp.sum(jnp.where(hit, i, 0), axis=1, keepdims=True)
    fsel_ref[...] = jnp.sum(jnp.where(hit, fall, 0.0), axis=1, keepdims=True)


def _perm(rank, fitness):
    return pl.pallas_call(
        _perm_body,
        grid=(NKEEP // ROWB,),
        in_specs=[
            pl.BlockSpec((1, N), lambda i: (0, 0)),
            pl.BlockSpec((1, N), lambda i: (0, 0)),
        ],
        out_specs=[
            pl.BlockSpec((ROWB, 1), lambda i: (i, 0)),
            pl.BlockSpec((ROWB, 1), lambda i: (i, 0)),
        ],
        out_shape=[
            jax.ShapeDtypeStruct((NKEEP, 1), jnp.int32),
            jax.ShapeDtypeStruct((NKEEP, 1), jnp.float32),
        ],
    )(rank.reshape(1, N), fitness.reshape(1, N))


def _scale_body(rows_ref, f_ref, o_ref):
    o_ref[...] = rows_ref[...] * f_ref[...]


def _scale(rows, fsel):
    return pl.pallas_call(
        _scale_body,
        grid=(NKEEP // EB if NKEEP >= EB else 1,),
        in_specs=[
            pl.BlockSpec((min(EB, NKEEP), F), lambda i: (i, 0)),
            pl.BlockSpec((min(EB, NKEEP), 1), lambda i: (i, 0)),
        ],
        out_specs=pl.BlockSpec((min(EB, NKEEP), F), lambda i: (i, 0)),
        out_shape=jax.ShapeDtypeStruct((NKEEP, F), jnp.float32),
    )(rows, fsel)


def kernel(node_features, batch, lin_W, lin_b, att_W, att_b, le1_W, le1_b, le2_W, le3_W, le3_b):
    x = node_features
    n = x.shape[0]
    d = jnp.sum(x * x, axis=1)
    nbr = _dist_topk(x, x.T, d)[:, :K]
    src = jnp.repeat(jnp.arange(n), K)
    dst = nbr.reshape(-1)
    x_src = x[src]
    x_q = jax.ops.segment_max(x_src, dst, num_segments=n)
    x_q = jnp.where(jnp.isfinite(x_q), x_q, 0.0)
    q = _q_matmul(x_q, lin_W, lin_b)
    score = _score(q[dst], x_src, att_W, att_b).reshape(-1)
    smax = jax.ops.segment_max(score, dst, num_segments=n)
    smax = jnp.where(jnp.isfinite(smax), smax, 0.0)
    e = _edge_exp(score.reshape(-1, 1), smax[dst].reshape(-1, 1)).reshape(-1)
    ssum = jax.ops.segment_sum(e, dst, num_segments=n)
    v_j = _vj(e.reshape(-1, 1), ssum[dst].reshape(-1, 1), x_src)
    x_new = jax.ops.segment_sum(v_j, dst, num_segments=n)
    a, b, xle3 = _le_matvecs(x_new, le1_W, le1_b, le2_W, le3_W)
    msg = _msg(a[dst], b[src])
    agg = jax.ops.segment_sum(msg, dst, num_segments=n)
    fitness = _fitness(agg, xle3, le3_b).reshape(-1)
    rank = _rank(fitness.reshape(-1, 1))
    perm, fsel = _perm(rank, fitness)
    perm = perm.reshape(-1)
    out = _scale(x_new[perm], fsel)
    new_batch = batch[perm]
    return out, new_batch


# speedup vs baseline: 2.2317x; 1.2026x over previous
import math
from functools import partial

import jax
import jax.numpy as jnp
from jax import lax
from jax.experimental import pallas as pl
from jax.experimental.pallas import tpu as pltpu, tpu_sc as plsc

N = 8192
F = 128
K = 10
NKEEP = 4096
ROWB = 256
EB = 2560
QB = EB // K


def _bf(v):
    return v.astype(jnp.bfloat16)


def _rep(blk, m):
    q = blk.shape[0]
    return jnp.broadcast_to(blk[:, None, :], (q, K, blk.shape[1])).reshape(q * K, blk.shape[1])


def _sc_gather(table, idx):
    E = idx.shape[0]
    info = plsc.get_sparse_core_info()
    nw = info.num_cores * info.num_subcores
    per = E // nw

    @partial(
        pl.kernel,
        mesh=plsc.VectorSubcoreMesh(core_axis_name="c", subcore_axis_name="s"),
        out_type=jax.ShapeDtypeStruct((E,), jnp.float32),
        scratch_types=[
            pltpu.VMEM((per,), jnp.int32),
            pltpu.VMEM((N,), jnp.float32),
            pltpu.VMEM((per,), jnp.float32),
        ],
    )
    def k(tab_hbm, idx_hbm, out_hbm, idx_v, tab_v, out_v):
        wid = lax.axis_index("s") * info.num_cores + lax.axis_index("c")
        base = wid * per
        pltpu.sync_copy(idx_hbm.at[pl.ds(base, per)], idx_v)
        pltpu.sync_copy(tab_hbm, tab_v)
        for i in range(per // 16):
            iv = idx_v[pl.ds(i * 16, 16)]
            out_v[pl.ds(i * 16, 16)] = plsc.load_gather(tab_v, [iv])
        pltpu.sync_copy(out_v, out_hbm.at[pl.ds(base, per)])

    return k(table, idx)


def _dist_topk_body(x_ref, xt_ref, dcol_ref, drow_ref, nbr_ref):
    g = jnp.dot(_bf(x_ref[...]), _bf(xt_ref[...]), preferred_element_type=jnp.float32)
    dist = dcol_ref[...] - 2.0 * g + drow_ref[...]
    lane = jax.lax.broadcasted_iota(jnp.int32, (ROWB, N), 1)
    for k in range(K):
        idx = jnp.argmin(dist, axis=1).astype(jnp.int32).reshape(ROWB, 1)
        nbr_ref[:, k : k + 1] = idx
        dist = jnp.where(lane == idx, jnp.inf, dist)
    nbr_ref[:, K:] = jnp.zeros((ROWB, 16 - K), jnp.int32)


def _dist_topk(x, xt, d):
    grid = (N // ROWB,)
    return pl.pallas_call(
        _dist_topk_body,
        grid=grid,
        in_specs=[
            pl.BlockSpec((ROWB, F), lambda i: (i, 0)),
            pl.BlockSpec((F, N), lambda i: (0, 0)),
            pl.BlockSpec((ROWB, 1), lambda i: (i, 0)),
            pl.BlockSpec((1, N), lambda i: (0, 0)),
        ],
        out_specs=pl.BlockSpec((ROWB, 16), lambda i: (i, 0)),
        out_shape=jax.ShapeDtypeStruct((N, 16), jnp.int32),
    )(x, xt, d.reshape(N, 1), d.reshape(1, N))


def _q_body(xq_ref, w_ref, b_ref, o_ref):
    o_ref[...] = (
        jnp.dot(_bf(xq_ref[...]), _bf(w_ref[...]), preferred_element_type=jnp.float32)
        + b_ref[...]
    )


def _q_matmul(x_q, lin_W, lin_b):
    return pl.pallas_call(
        _q_body,
        grid=(8,),
        in_specs=[
            pl.BlockSpec((N // 8, F), lambda i: (i, 0)),
            pl.BlockSpec((F, F), lambda i: (0, 0)),
            pl.BlockSpec((1, F), lambda i: (0, 0)),
        ],
        out_specs=pl.BlockSpec((N // 8, F), lambda i: (i, 0)),
        out_shape=jax.ShapeDtypeStruct((N, F), jnp.float32),
    )(x_q, lin_W, lin_b.reshape(1, F))


def _score_body(qd_ref, x_ref, w_ref, b_ref, o_ref):
    ef = jnp.concatenate([qd_ref[...], _rep(x_ref[...], F)], axis=1)
    s = jnp.dot(_bf(ef), _bf(w_ref[...]), preferred_element_type=jnp.float32) + b_ref[...]
    o_ref[...] = jnp.where(s >= 0, s, 0.2 * s)


def _score(q_dst, x, att_W, att_b):
    E = N * K
    return pl.pallas_call(
        _score_body,
        grid=(E // EB,),
        in_specs=[
            pl.BlockSpec((EB, F), lambda i: (i, 0)),
            pl.BlockSpec((QB, F), lambda i: (i, 0)),
            pl.BlockSpec((2 * F, 1), lambda i: (0, 0)),
            pl.BlockSpec((1, 1), lambda i: (0, 0)),
        ],
        out_specs=pl.BlockSpec((EB, 1), lambda i: (i, 0)),
        out_shape=jax.ShapeDtypeStruct((E, 1), jnp.float32),
    )(q_dst, x, att_W, att_b.reshape(1, 1))


def _exp_body(s_ref, m_ref, o_ref):
    o_ref[...] = jnp.exp(s_ref[...] - m_ref[...])


def _edge_exp(score, smax_d):
    E = N * K
    return pl.pallas_call(
        _exp_body,
        grid=(E // EB,),
        in_specs=[
            pl.BlockSpec((EB, 1), lambda i: (i, 0)),
            pl.BlockSpec((EB, 1), lambda i: (i, 0)),
        ],
        out_specs=pl.BlockSpec((EB, 1), lambda i: (i, 0)),
        out_shape=jax.ShapeDtypeStruct((E, 1), jnp.float32),
    )(score, smax_d)


def _vj_body(e_ref, sd_ref, x_ref, o_ref):
    w = e_ref[...] / (sd_ref[...] + 1e-16)
    o_ref[...] = _rep(x_ref[...], F) * w


def _vj(e, ssum_d, x):
    E = N * K
    return pl.pallas_call(
        _vj_body,
        grid=(E // EB,),
        in_specs=[
            pl.BlockSpec((EB, 1), lambda i: (i, 0)),
            pl.BlockSpec((EB, 1), lambda i: (i, 0)),
            pl.BlockSpec((QB, F), lambda i: (i, 0)),
        ],
        out_specs=pl.BlockSpec((EB, F), lambda i: (i, 0)),
        out_shape=jax.ShapeDtypeStruct((E, F), jnp.float32),
    )(e, ssum_d, x)


def _le_body(xn_ref, w1_ref, b1_ref, w2_ref, w3_ref, a_ref, b_ref, c_ref):
    xb = _bf(xn_ref[...])
    a_ref[...] = jnp.dot(xb, _bf(w1_ref[...]), preferred_element_type=jnp.float32) + b1_ref[...]
    b_ref[...] = jnp.dot(xb, _bf(w2_ref[...]), preferred_element_type=jnp.float32)
    c_ref[...] = jnp.dot(xb, _bf(w3_ref[...]), preferred_element_type=jnp.float32)


def _le_matvecs(x_new, le1_W, le1_b, le2_W, le3_W):
    return pl.pallas_call(
        _le_body,
        grid=(8,),
        in_specs=[
            pl.BlockSpec((N // 8, F), lambda i: (i, 0)),
            pl.BlockSpec((F, 1), lambda i: (0, 0)),
            pl.BlockSpec((1, 1), lambda i: (0, 0)),
            pl.BlockSpec((F, 1), lambda i: (0, 0)),
            pl.BlockSpec((F, 1), lambda i: (0, 0)),
        ],
        out_specs=[
            pl.BlockSpec((N // 8, 1), lambda i: (i, 0)),
            pl.BlockSpec((N // 8, 1), lambda i: (i, 0)),
            pl.BlockSpec((N // 8, 1), lambda i: (i, 0)),
        ],
        out_shape=[
            jax.ShapeDtypeStruct((N, 1), jnp.float32),
            jax.ShapeDtypeStruct((N, 1), jnp.float32),
            jax.ShapeDtypeStruct((N, 1), jnp.float32),
        ],
    )(x_new, le1_W, le1_b.reshape(1, 1), le2_W, le3_W)


def _msg_body(ad_ref, b_ref, o_ref):
    o_ref[...] = ad_ref[...] - _rep(b_ref[...], 1)


def _msg(a_d, b):
    E = N * K
    return pl.pallas_call(
        _msg_body,
        grid=(E // EB,),
        in_specs=[
            pl.BlockSpec((EB, 1), lambda i: (i, 0)),
            pl.BlockSpec((QB, 1), lambda i: (i, 0)),
        ],
        out_specs=pl.BlockSpec((EB, 1), lambda i: (i, 0)),
        out_shape=jax.ShapeDtypeStruct((E, 1), jnp.float32),
    )(a_d, b)


def _fit_body(agg_ref, c_ref, b3_ref, o_ref):
    o_ref[...] = jax.nn.sigmoid((agg_ref[...] + c_ref[...]) + b3_ref[...])


def _fitness(agg, xle3, le3_b):
    return pl.pallas_call(
        _fit_body,
        grid=(8,),
        in_specs=[
            pl.BlockSpec((N // 8, 1), lambda i: (i, 0)),
            pl.BlockSpec((N // 8, 1), lambda i: (i, 0)),
            pl.BlockSpec((1, 1), lambda i: (0, 0)),
        ],
        out_specs=pl.BlockSpec((N // 8, 1), lambda i: (i, 0)),
        out_shape=jax.ShapeDtypeStruct((N, 1), jnp.float32),
    )(agg, xle3, le3_b.reshape(1, 1))


def _rank_body(fblk_ref, fall_ref, rank_ref):
    fb = fblk_ref[...]
    fa = fall_ref[...]
    i_blk = (
        jax.lax.broadcasted_iota(jnp.int32, (ROWB, 1), 0)
        + pl.program_id(0) * ROWB
    )
    j = jax.lax.broadcasted_iota(jnp.int32, (ROWB, N), 1)
    gt = (fa > fb).astype(jnp.int32)
    eq = ((fa == fb) & (j < i_blk)).astype(jnp.int32)
    rank_ref[...] = jnp.sum(gt + eq, axis=1, keepdims=True)


def _rank(fitness):
    return pl.pallas_call(
        _rank_body,
        grid=(N // ROWB,),
        in_specs=[
            pl.BlockSpec((ROWB, 1), lambda i: (i, 0)),
            pl.BlockSpec((1, N), lambda i: (0, 0)),
        ],
        out_specs=pl.BlockSpec((ROWB, 1), lambda i: (i, 0)),
        out_shape=jax.ShapeDtypeStruct((N, 1), jnp.int32),
    )(fitness, fitness.reshape(1, N))


def _perm_body(rank_ref, fall_ref, perm_ref, fsel_ref):
    r_blk = (
        jax.lax.broadcasted_iota(jnp.int32, (ROWB, 1), 0)
        + pl.program_id(0) * ROWB
    )
    ranks = rank_ref[...]
    fall = fall_ref[...]
    i = jax.lax.broadcasted_iota(jnp.int32, (ROWB, N), 1)
    hit = ranks == r_blk
    perm_ref[...] = jnp.sum(jnp.where(hit, i, 0), axis=1, keepdims=True)
    fsel_ref[...] = jnp.sum(jnp.where(hit, fall, 0.0), axis=1, keepdims=True)


def _perm(rank, fitness):
    return pl.pallas_call(
        _perm_body,
        grid=(NKEEP // ROWB,),
        in_specs=[
            pl.BlockSpec((1, N), lambda i: (0, 0)),
            pl.BlockSpec((1, N), lambda i: (0, 0)),
        ],
        out_specs=[
            pl.BlockSpec((ROWB, 1), lambda i: (i, 0)),
            pl.BlockSpec((ROWB, 1), lambda i: (i, 0)),
        ],
        out_shape=[
            jax.ShapeDtypeStruct((NKEEP, 1), jnp.int32),
            jax.ShapeDtypeStruct((NKEEP, 1), jnp.float32),
        ],
    )(rank.reshape(1, N), fitness.reshape(1, N))


def _scale_body(rows_ref, f_ref, o_ref):
    o_ref[...] = rows_ref[...] * f_ref[...]


def _scale(rows, fsel):
    return pl.pallas_call(
        _scale_body,
        grid=(2,),
        in_specs=[
            pl.BlockSpec((NKEEP // 2, F), lambda i: (i, 0)),
            pl.BlockSpec((NKEEP // 2, 1), lambda i: (i, 0)),
        ],
        out_specs=pl.BlockSpec((NKEEP // 2, F), lambda i: (i, 0)),
        out_shape=jax.ShapeDtypeStruct((NKEEP, F), jnp.float32),
    )(rows, fsel)


def kernel(node_features, batch, lin_W, lin_b, att_W, att_b, le1_W, le1_b, le2_W, le3_W, le3_b):
    x = node_features
    n = x.shape[0]
    d = jnp.sum(x * x, axis=1)
    nbr = _dist_topk(x, x.T, d)[:, :K]
    dst = nbr.reshape(-1)
    x_src = jnp.repeat(x, K, axis=0)
    x_q = jax.ops.segment_max(x_src, dst, num_segments=n)
    x_q = jnp.where(jnp.isfinite(x_q), x_q, 0.0)
    q = _q_matmul(x_q, lin_W, lin_b)
    score = _score(q[dst], x, att_W, att_b).reshape(-1)
    smax = jax.ops.segment_max(score, dst, num_segments=n)
    smax = jnp.where(jnp.isfinite(smax), smax, 0.0)
    e = _edge_exp(score.reshape(-1, 1), smax[dst].reshape(-1, 1)).reshape(-1)
    ssum = jax.ops.segment_sum(e, dst, num_segments=n)
    v_j = _vj(e.reshape(-1, 1), ssum[dst].reshape(-1, 1), x)
    x_new = jax.ops.segment_sum(v_j, dst, num_segments=n)
    a, b, xle3 = _le_matvecs(x_new, le1_W, le1_b, le2_W, le3_W)
    msg = _msg(a[dst], b)
    agg = jax.ops.segment_sum(msg, dst, num_segments=n)
    fitness = _fitness(agg, xle3, le3_b).reshape(-1)
    rank = _rank(fitness.reshape(-1, 1))
    perm, fsel = _perm(rank, fitness)
    perm = perm.reshape(-1)
    out = _scale(x_new[perm], fsel)
    new_batch = batch[perm]
    return out, new_batch
